# baseline (speedup 1.0000x reference)
import functools

import jax
import jax.numpy as jnp
from jax.experimental import pallas as pl
from jax.experimental.pallas import tpu as pltpu

_B, _K, _C, _H, _W = 2, 3, 3, 256, 256
_RATE = 0.5
_ENC = [(3, 8, 5, 1, 2), (8, 16, 3, 2, 1), (16, 32, 3, 2, 1), (32, 64, 3, 2, 1), (64, 64, 3, 2, 1)]
_DEC = [('skip1', 64, 64, 1, 1, 0, True, 'lrelu'), ('skip2', 32, 64, 1, 1, 0, True, 'lrelu'), ('conv1', 128, 64, 3, 1, 1, True, 'lrelu'), ('conv2', 128, 32, 3, 1, 1, True, 'lrelu'), ('conv3', 32, 16, 3, 1, 1, True, 'lrelu'), ('conv4', 16, 8, 3, 1, 1, True, 'lrelu'), ('conv5', 8, 3, 5, 1, 2, False, 'tanh')]


def _conv_block(x, w, b, g, be, stride, pad, norm=True, act='lrelu'):
    if pad > 0:
        x = jnp.pad(x, ((0, 0), (0, 0), (pad, pad), (pad, pad)), mode='reflect')
    x = jax.lax.conv_general_dilated(x, w, (stride, stride), 'VALID',
                                     dimension_numbers=('NCHW', 'OIHW', 'NCHW'))
    x = x + b[None, :, None, None]
    if norm:
        m = x.mean(axis=(0, 2, 3), keepdims=True)
        v = x.var(axis=(0, 2, 3), keepdims=True)
        x = (x - m) / jnp.sqrt(v + 1e-5)
        x = x * g[None, :, None, None] + be[None, :, None, None]
    if act == 'lrelu':
        x = jnp.where(x >= 0, x, 0.2 * x)
    elif act == 'tanh':
        x = jnp.tanh(x)
    return x


def _up2(x):
    return jnp.repeat(jnp.repeat(x, 2, axis=2), 2, axis=3)


def _fusion_body(feat_ref, refs_ref, fi_ref, sim_ref, out_ref, *, n, hw, num):
    b_idx = pl.program_id(0)
    featT = feat_ref[0]
    fi = fi_ref[0]

    fnorm = jnp.sqrt(jnp.sum(featT * featT, axis=1, keepdims=True))
    wfT = featT / jnp.maximum(fnorm, 1e-12)

    hidx = jax.lax.broadcasted_iota(jnp.int32, (num, hw), 1)
    GT = (hidx == fi).astype(jnp.float32)

    feat_selT = jax.lax.dot(GT, featT)
    wfsT = jax.lax.dot(GT, wfT)
    wnorm = jnp.sqrt(jnp.sum(wfsT * wfsT, axis=1, keepdims=True))
    wfsT = wfsT / jnp.maximum(wnorm, 1e-12)

    base_sim = sim_ref[b_idx, 0]
    acc = base_sim * feat_selT
    hiota = jax.lax.broadcasted_iota(jnp.int32, (num, hw), 1)
    for j in range(n):
        refT = refs_ref[0, j]
        rnorm = jnp.sqrt(jnp.sum(refT * refT, axis=1, keepdims=True))
        wrT = refT / jnp.maximum(rnorm, 1e-12)
        fx = jax.lax.dot_general(wfsT, wrT, (((1,), (1,)), ((), ())))
        maxv = jnp.max(fx, axis=1, keepdims=True)
        eligible = fx >= maxv
        ind = jnp.min(jnp.where(eligible, hiota, hw), axis=1, keepdims=True)
        onehot = (hiota == ind).astype(jnp.float32)
        ref_selT = jax.lax.dot(onehot, refT)
        acc = acc + sim_ref[b_idx, 1 + j] * ref_selT

    covered = jax.lax.dot_general(GT, jnp.ones((num, 1), jnp.float32),
                                  (((0,), (0,)), ((), ())))
    scattered = jax.lax.dot_general(GT, acc, (((0,), (0,)), ((), ())))
    out_ref[0] = featT * (1.0 - covered) + scattered


def _fusion_pallas(featT, refsT, fi, sim):
    b, hw, c = featT.shape
    n = refsT.shape[1]
    num = fi.shape[1]
    body = functools.partial(_fusion_body, n=n, hw=hw, num=num)
    return pl.pallas_call(
        body,
        grid=(b,),
        in_specs=[
            pl.BlockSpec((1, hw, c), lambda i: (i, 0, 0)),
            pl.BlockSpec((1, n, hw, c), lambda i: (i, 0, 0, 0)),
            pl.BlockSpec((1, num, 1), lambda i: (i, 0, 0)),
            pl.BlockSpec(memory_space=pltpu.SMEM),
        ],
        out_specs=pl.BlockSpec((1, hw, c), lambda i: (i, 0, 0)),
        out_shape=jax.ShapeDtypeStruct((b, hw, c), jnp.float32),
    )(featT, refsT, fi, sim)


def _fusion(feat, refs_all, similarity, key):
    b, kk, c, h, w = refs_all.shape
    n = kk - 1
    hw = h * w
    num = int(_RATE * hw)
    idx_keys = jax.random.split(key, b)
    feat_indices = jnp.stack(
        [jax.random.permutation(idx_keys[i], hw)[:num] for i in range(b)])

    featT = feat.reshape(b, c, hw).transpose(0, 2, 1)
    refsT = refs_all[:, 1:].reshape(b, n, c, hw).transpose(0, 1, 3, 2)
    fi = feat_indices.astype(jnp.int32).reshape(b, num, 1)
    outT = _fusion_pallas(featT, refsT, fi, similarity)
    return outT.transpose(0, 2, 1).reshape(b, c, h, w)


def kernel(xs, params):
    b, k, cc, hh, ww = xs.shape
    x = xs.reshape(b * k, cc, hh, ww)
    feats = []
    for li, (ci, co, kk, st, pd) in enumerate(_ENC):
        x = _conv_block(x, params['enc%d_w' % li], params['enc%d_b' % li],
                        params['enc%d_g' % li], params['enc%d_be' % li], st, pd)
        feats.append(x)
    x5 = feats[-1]
    c, h, w = x5.shape[1], x5.shape[2], x5.shape[3]
    querys5 = x5.reshape(b, k, c, h, w)
    sim = jax.random.uniform(jax.random.key(42), (b, k), jnp.float32)
    sim = sim / jnp.sum(sim, axis=1, keepdims=True)
    feat_gen = _fusion(querys5[:, 0], querys5, sim, jax.random.key(7))
    skips = [f.reshape(b, k, f.shape[1], f.shape[2], f.shape[3])[:, 0]
             for f in feats[:-1]]
    x = _up2(feat_gen)
    s4 = _conv_block(skips[3], params['skip1_w'], params['skip1_b'],
                     params['skip1_g'], params['skip1_be'], 1, 0)
    x = jnp.concatenate([x, s4], axis=1)
    x = _conv_block(x, params['conv1_w'], params['conv1_b'],
                    params['conv1_g'], params['conv1_be'], 1, 1)
    x = _up2(x)
    s3 = _conv_block(skips[2], params['skip2_w'], params['skip2_b'],
                     params['skip2_g'], params['skip2_be'], 1, 0)
    x = jnp.concatenate([x, s3], axis=1)
    x = _conv_block(x, params['conv2_w'], params['conv2_b'],
                    params['conv2_g'], params['conv2_be'], 1, 1)
    x = _up2(x)
    x = _conv_block(x, params['conv3_w'], params['conv3_b'],
                    params['conv3_g'], params['conv3_be'], 1, 1)
    x = _up2(x)
    x = _conv_block(x, params['conv4_w'], params['conv4_b'],
                    params['conv4_g'], params['conv4_be'], 1, 1)
    x = _conv_block(x, params['conv5_w'], params['conv5_b'],
                    None, None, 1, 2, norm=False, act='tanh')
    return x
